# baseline (speedup 1.0000x reference)
import functools

import jax
import jax.numpy as jnp
from jax import lax
from jax.experimental import pallas as pl
from jax.experimental.pallas import tpu as pltpu
from jax.experimental.pallas import tpu_sc as plsc

NC = 2
NS = 16
LANES = 16
CHUNK = 128


def _sc_aggregate(x2, gidx, dstp, zeros, ones, n_rows, h):
    ep = dstp.shape[0]
    epw = ep // NS
    nchunks = epw // CHUNK
    half = nchunks // 2
    rpt = n_rows // NS

    mesh = plsc.VectorSubcoreMesh(core_axis_name="c", subcore_axis_name="s")

    @functools.partial(
        pl.kernel,
        out_type=[
            jax.ShapeDtypeStruct((NC * n_rows, h), jnp.float32),
            jax.ShapeDtypeStruct((NC * n_rows, h), jnp.float32),
        ],
        mesh=mesh,
    scratch_types=[
            pltpu.VMEM_SHARED((n_rows, h), jnp.float32),
            pltpu.VMEM((CHUNK,), jnp.int32),
            pltpu.VMEM((CHUNK,), jnp.int32),
            pltpu.VMEM((CHUNK // LANES, LANES), jnp.int32),
            pltpu.VMEM((CHUNK, h), jnp.float32),
            pltpu.VMEM((CHUNK, h), jnp.float32),
            pltpu.SemaphoreType.DMA,
        ],
    )
    def agg(x2_h, gidx_h, dst_h, zeros_h, ones_h,
            sum_out, cnt_out,
            acc, gidx_v, dst_v, dst2_v, rows_v, ones_v, sem):
        c = lax.axis_index("c")
        s = lax.axis_index("s")

        gbase = c * ep + s * epw
        ebase = s * epw
        obase = c * n_rows + s * rpt

        pltpu.sync_copy(zeros_h, ones_v)

        def zbody(k, carry):
            pltpu.sync_copy(ones_v, acc.at[pl.ds(s * rpt + k * CHUNK, CHUNK)])
            return carry

        lax.fori_loop(0, rpt // CHUNK, zbody, 0)
        plsc.subcore_barrier()

        def body(k, carry):
            pltpu.sync_copy(gidx_h.at[pl.ds(gbase + k * CHUNK, CHUNK)], gidx_v)
            pltpu.sync_copy(dst_h.at[pl.ds(ebase + k * CHUNK, CHUNK)], dst_v)
            for j in range(CHUNK // LANES):
                dst2_v[j, :] = dst_v[pl.ds(j * LANES, LANES)]
            pltpu.async_copy(x2_h.at[gidx_v], rows_v, sem).wait()
            for j in range(CHUNK // LANES):
                pltpu.sync_copy(rows_v.at[pl.ds(j * LANES, LANES)],
                                acc.at[dst2_v.at[j]], add=True)
            return carry

        lax.fori_loop(0, nchunks, body, 0)
        plsc.subcore_barrier()

        def wbody(k, carry):
            row = s * rpt + k * CHUNK
            pltpu.sync_copy(acc.at[pl.ds(row, CHUNK)], rows_v)
            pltpu.sync_copy(rows_v, sum_out.at[pl.ds(obase + k * CHUNK, CHUNK)])
            pltpu.sync_copy(ones_v, acc.at[pl.ds(row, CHUNK)])
            return carry

        lax.fori_loop(0, rpt // CHUNK, wbody, 0)
        pltpu.sync_copy(ones_h, ones_v)
        plsc.subcore_barrier()

        def cbody(k, carry):
            base = ebase + (c * half + k) * CHUNK
            pltpu.sync_copy(dst_h.at[pl.ds(base, CHUNK)], dst_v)
            for j in range(CHUNK // LANES):
                dst2_v[j, :] = dst_v[pl.ds(j * LANES, LANES)]
            for j in range(CHUNK // LANES):
                pltpu.sync_copy(ones_v.at[pl.ds(0, LANES)],
                                acc.at[dst2_v.at[j]], add=True)
            return carry

        lax.fori_loop(0, half, cbody, 0)
        plsc.subcore_barrier()

        def cwbody(k, carry):
            row = s * rpt + k * CHUNK
            pltpu.sync_copy(acc.at[pl.ds(row, CHUNK)], rows_v)
            pltpu.sync_copy(rows_v, cnt_out.at[pl.ds(obase + k * CHUNK, CHUNK)])
            return carry

        lax.fori_loop(0, rpt // CHUNK, cwbody, 0)

    return agg(x2, gidx, dstp, zeros, ones)


def _dense_body(s0_ref, s1_ref, c0_ref, c1_ref, x_ref, wl_ref, wr_ref,
                b_ref, o_ref):
    cnt = jnp.maximum(c0_ref[:, 0:1] + c1_ref[:, 0:1], 1.0)
    mean = jnp.concatenate([s0_ref[...], s1_ref[...]], axis=1) / cnt
    acc = jnp.dot(mean, wl_ref[...], preferred_element_type=jnp.float32)
    acc = acc + jnp.dot(x_ref[...], wr_ref[...],
                        preferred_element_type=jnp.float32)
    acc = acc + b_ref[0:1, :]
    o_ref[...] = jnp.maximum(acc, 0.0)


def _tc_dense(sum0, sum1, cnt0, cnt1, x, wl_t, wr_t, b8):
    n, d = x.shape
    h = sum0.shape[1]
    blk = 1000
    grid = n // blk
    return pl.pallas_call(
        _dense_body,
        grid=(grid,),
        in_specs=[
            pl.BlockSpec((blk, h), lambda i: (i, 0)),
            pl.BlockSpec((blk, h), lambda i: (i, 0)),
            pl.BlockSpec((blk, h), lambda i: (i, 0)),
            pl.BlockSpec((blk, h), lambda i: (i, 0)),
            pl.BlockSpec((blk, d), lambda i: (i, 0)),
            pl.BlockSpec((d, d), lambda i: (0, 0)),
            pl.BlockSpec((d, d), lambda i: (0, 0)),
            pl.BlockSpec((8, d), lambda i: (0, 0)),
        ],
        out_specs=pl.BlockSpec((blk, d), lambda i: (i, 0)),
        out_shape=jax.ShapeDtypeStruct((n, d), jnp.float32),
    )(sum0, sum1, cnt0, cnt1, x, wl_t, wr_t, b8)


def kernel(x, edge_index, W_l, b_l, W_r):
    n_nodes, d = x.shape
    h = d // 2
    e = edge_index.shape[1]
    src = edge_index[0].astype(jnp.int32)
    dst = edge_index[1].astype(jnp.int32)

    step = NS * CHUNK
    ep = ((e + 2 * step - 1) // (2 * step)) * (2 * step)
    n_rows = ((n_nodes + 16 + step - 1) // step) * step
    pad = ep - e
    if pad:
        ar = jnp.arange(pad, dtype=jnp.int32)
        src = jnp.concatenate([src, (ar * 7919) % n_nodes])
        dst = jnp.concatenate([dst, n_nodes + ar % (n_rows - n_nodes)])

    x2 = x.reshape(2 * n_nodes, h)
    gidx = jnp.concatenate([src * 2, src * 2 + 1])
    zeros = jnp.zeros((CHUNK, h), jnp.float32)
    ones = jnp.ones((CHUNK, h), jnp.float32)

    sum_cat, cnt_cat = _sc_aggregate(x2, gidx, dst, zeros, ones, n_rows, h)
    sum0 = sum_cat[:n_nodes]
    sum1 = sum_cat[n_rows:n_rows + n_nodes]
    cnt0 = cnt_cat[:n_nodes]
    cnt1 = cnt_cat[n_rows:n_rows + n_nodes]
    return _tc_dense(sum0, sum1, cnt0, cnt1, x, W_l.T, W_r.T,
                     jnp.broadcast_to(b_l, (8, d)))


# speedup vs baseline: 1.4719x; 1.4719x over previous
import functools

import jax
import jax.numpy as jnp
from jax import lax
from jax.experimental import pallas as pl
from jax.experimental.pallas import tpu as pltpu
from jax.experimental.pallas import tpu_sc as plsc

NC = 2
NS = 16
LANES = 16
CHUNK = 128


def _sc_aggregate(x2, gidx2, dstp, zeros, ones, n_rows, h):
    ep = dstp.shape[0]
    epw = ep // NS
    nchunks = epw // CHUNK
    half = nchunks // 2
    nstg = 2
    sch = nchunks // nstg
    rpt = n_rows // NS

    mesh = plsc.VectorSubcoreMesh(core_axis_name="c", subcore_axis_name="s")

    @functools.partial(
        pl.kernel,
        out_type=[
            jax.ShapeDtypeStruct((NC * n_rows, h), jnp.float32),
            jax.ShapeDtypeStruct((NC * n_rows, h), jnp.float32),
        ],
        mesh=mesh,
    scratch_types=[
            pltpu.VMEM_SHARED((n_rows, h), jnp.float32),
            pltpu.VMEM((sch, 1, CHUNK), jnp.int32),
            pltpu.VMEM((CHUNK,), jnp.int32),
            pltpu.VMEM((CHUNK,), jnp.int32),
            pltpu.VMEM((CHUNK // LANES, LANES), jnp.int32),
            pltpu.VMEM((CHUNK, h), jnp.float32),
            pltpu.VMEM((CHUNK, h), jnp.float32),
            pltpu.SemaphoreType.DMA,
            pltpu.SemaphoreType.DMA,
        ],
    )
    def agg(x2_h, gidx2_h, dst_h, zeros_h, ones_h,
            sum_out, cnt_out,
            acc, gidx_all, dva, dvb, dst2_v, rows_a, rows_b, sem_a, sem_b):
        c = lax.axis_index("c")
        s = lax.axis_index("s")

        gtile = (c * ep + s * epw) // CHUNK
        ebase = s * epw
        obase = c * n_rows + s * rpt

        def scat(rows_v, dv):
            for j in range(CHUNK // LANES):
                dst2_v[j, :] = dv[pl.ds(j * LANES, LANES)]
            for j in range(CHUNK // LANES):
                pltpu.sync_copy(rows_v.at[pl.ds(j * LANES, LANES)],
                                acc.at[dst2_v.at[j]], add=True)

        pltpu.sync_copy(zeros_h, rows_a)

        def zbody(k, carry):
            pltpu.sync_copy(rows_a, acc.at[pl.ds(s * rpt + k * CHUNK, CHUNK)])
            return carry

        lax.fori_loop(0, rpt // CHUNK, zbody, 0)
        plsc.subcore_barrier()

        for st in range(nstg):
            pltpu.sync_copy(gidx2_h.at[pl.ds(gtile + st * sch, sch)], gidx_all)
            base_e = ebase + st * sch * CHUNK
            pltpu.sync_copy(dst_h.at[pl.ds(base_e, CHUNK)], dva)
            pltpu.async_copy(x2_h.at[gidx_all.at[0, 0]], rows_a, sem_a)

            def pair(i, carry):
                k0 = 2 * i
                pltpu.sync_copy(
                    dst_h.at[pl.ds(base_e + (k0 + 1) * CHUNK, CHUNK)], dvb)
                pltpu.async_copy(x2_h.at[gidx_all.at[k0 + 1, 0]], rows_b, sem_b)
                pltpu.make_async_copy(
                    x2_h.at[gidx_all.at[k0, 0]], rows_a, sem_a).wait()
                scat(rows_a, dva)

                @pl.when(k0 + 2 < sch)
                def _():
                    pltpu.sync_copy(
                        dst_h.at[pl.ds(base_e + (k0 + 2) * CHUNK, CHUNK)], dva)
                    pltpu.async_copy(
                        x2_h.at[gidx_all.at[k0 + 2, 0]], rows_a, sem_a)

                pltpu.make_async_copy(
                    x2_h.at[gidx_all.at[k0 + 1, 0]], rows_b, sem_b).wait()
                scat(rows_b, dvb)
                return carry

            lax.fori_loop(0, sch // 2, pair, 0)
        plsc.subcore_barrier()

        pltpu.sync_copy(zeros_h, rows_a)

        def wbody(k, carry):
            row = s * rpt + k * CHUNK
            pltpu.sync_copy(acc.at[pl.ds(row, CHUNK)], rows_b)
            pltpu.sync_copy(rows_b, sum_out.at[pl.ds(obase + k * CHUNK, CHUNK)])
            pltpu.sync_copy(rows_a, acc.at[pl.ds(row, CHUNK)])
            return carry

        lax.fori_loop(0, rpt // CHUNK, wbody, 0)
        pltpu.sync_copy(ones_h, rows_a)
        plsc.subcore_barrier()

        cbase = ebase + c * half * CHUNK
        pltpu.sync_copy(dst_h.at[pl.ds(cbase, CHUNK)], dva)

        def cpair(i, carry):
            k0 = 2 * i
            pltpu.async_copy(
                dst_h.at[pl.ds(cbase + (k0 + 1) * CHUNK, CHUNK)], dvb, sem_a)
            scat(rows_a, dva)
            pltpu.make_async_copy(
                dst_h.at[pl.ds(cbase + (k0 + 1) * CHUNK, CHUNK)], dvb,
                sem_a).wait()

            @pl.when(k0 + 2 < half)
            def _():
                pltpu.async_copy(
                    dst_h.at[pl.ds(cbase + (k0 + 2) * CHUNK, CHUNK)], dva,
                    sem_b)

            scat(rows_a, dvb)

            @pl.when(k0 + 2 < half)
            def _():
                pltpu.make_async_copy(
                    dst_h.at[pl.ds(cbase + (k0 + 2) * CHUNK, CHUNK)], dva,
                    sem_b).wait()

            return carry

        lax.fori_loop(0, half // 2, cpair, 0)
        plsc.subcore_barrier()

        def cwbody(k, carry):
            row = s * rpt + k * CHUNK
            pltpu.sync_copy(acc.at[pl.ds(row, CHUNK)], rows_b)
            pltpu.sync_copy(rows_b, cnt_out.at[pl.ds(obase + k * CHUNK, CHUNK)])
            return carry

        lax.fori_loop(0, rpt // CHUNK, cwbody, 0)

    return agg(x2, gidx2, dstp, zeros, ones)


def _dense_body(s0_ref, s1_ref, c0_ref, c1_ref, x_ref, wl_ref, wr_ref,
                b_ref, o_ref):
    cnt = jnp.maximum(c0_ref[:, 0:1] + c1_ref[:, 0:1], 1.0)
    mean = jnp.concatenate([s0_ref[...], s1_ref[...]], axis=1) / cnt
    acc = jnp.dot(mean, wl_ref[...], preferred_element_type=jnp.float32)
    acc = acc + jnp.dot(x_ref[...], wr_ref[...],
                        preferred_element_type=jnp.float32)
    acc = acc + b_ref[0:1, :]
    o_ref[...] = jnp.maximum(acc, 0.0)


def _tc_dense(sum0, sum1, cnt0, cnt1, x, wl_t, wr_t, b8):
    n, d = x.shape
    h = sum0.shape[1]
    blk = 1000
    grid = n // blk
    return pl.pallas_call(
        _dense_body,
        grid=(grid,),
        in_specs=[
            pl.BlockSpec((blk, h), lambda i: (i, 0)),
            pl.BlockSpec((blk, h), lambda i: (i, 0)),
            pl.BlockSpec((blk, h), lambda i: (i, 0)),
            pl.BlockSpec((blk, h), lambda i: (i, 0)),
            pl.BlockSpec((blk, d), lambda i: (i, 0)),
            pl.BlockSpec((d, d), lambda i: (0, 0)),
            pl.BlockSpec((d, d), lambda i: (0, 0)),
            pl.BlockSpec((8, d), lambda i: (0, 0)),
        ],
        out_specs=pl.BlockSpec((blk, d), lambda i: (i, 0)),
        out_shape=jax.ShapeDtypeStruct((n, d), jnp.float32),
    )(sum0, sum1, cnt0, cnt1, x, wl_t, wr_t, b8)


def kernel(x, edge_index, W_l, b_l, W_r):
    n_nodes, d = x.shape
    h = d // 2
    e = edge_index.shape[1]
    src = edge_index[0].astype(jnp.int32)
    dst = edge_index[1].astype(jnp.int32)

    step = NS * CHUNK
    ep = ((e + 2 * step - 1) // (2 * step)) * (2 * step)
    n_rows = ((n_nodes + 16 + step - 1) // step) * step
    pad = ep - e
    if pad:
        ar = jnp.arange(pad, dtype=jnp.int32)
        src = jnp.concatenate([src, (ar * 7919) % n_nodes])
        dst = jnp.concatenate([dst, n_nodes + ar % (n_rows - n_nodes)])

    x2 = x.reshape(2 * n_nodes, h)
    gidx2 = jnp.concatenate([src * 2, src * 2 + 1]).reshape(-1, 1, CHUNK)
    zeros = jnp.zeros((CHUNK, h), jnp.float32)
    ones = jnp.ones((CHUNK, h), jnp.float32)

    sum_cat, cnt_cat = _sc_aggregate(x2, gidx2, dst, zeros, ones, n_rows, h)
    sum0 = sum_cat[:n_nodes]
    sum1 = sum_cat[n_rows:n_rows + n_nodes]
    cnt0 = cnt_cat[:n_nodes]
    cnt1 = cnt_cat[n_rows:n_rows + n_nodes]
    return _tc_dense(sum0, sum1, cnt0, cnt1, x, W_l.T, W_r.T,
                     jnp.broadcast_to(b_l, (8, d)))
